# baseline (speedup 1.0000x reference)
import functools

import jax
import jax.numpy as jnp
from jax import lax
from jax.experimental import pallas as pl
from jax.experimental.pallas import tpu as pltpu
from jax.experimental.pallas import tpu_sc as plsc

_B = 16384
_D = 32
_NC = 2
_NS = 16
_NW = _NC * _NS
_BPW = _B // _NW
_CH = 16
_NCHK = _BPW // _CH
_NBUF = 2


def _sc_body(uidx_hbm, iidx_hbm, emb_u_hbm, emb_i_hbm, out_hbm,
             uidx_v, iidx_v, tidx_u, tidx_i, slab_u, slab_i, dot_v,
             sems_u, sems_i):
    c = lax.axis_index("c")
    s = lax.axis_index("s")
    wid = s * _NC + c
    base = wid * _BPW
    pltpu.sync_copy(uidx_hbm.at[pl.ds(base, _BPW)], uidx_v)
    pltpu.sync_copy(iidx_hbm.at[pl.ds(base, _BPW)], iidx_v)

    lanes = lax.iota(jnp.int32, 16)

    def tiles_body(i, _):
        tidx_u[pl.ds(i * 16, 16)] = lax.div(uidx_v[pl.ds(i * 16, 16)], 8)
        tidx_i[pl.ds(i * 16, 16)] = lax.div(iidx_v[pl.ds(i * 16, 16)], 8)
        return ()

    lax.fori_loop(0, _BPW // 16, tiles_body, ())

    def fire(k, b):
        pltpu.async_copy(emb_u_hbm.at[tidx_u.at[pl.ds(k * _CH, _CH)]],
                         slab_u.at[b], sems_u[b])
        pltpu.async_copy(emb_i_hbm.at[tidx_i.at[pl.ds(k * _CH, _CH)]],
                         slab_i.at[b], sems_i[b])

    def drain(b):
        pltpu.make_async_copy(emb_u_hbm.at[pl.ds(0, _CH)],
                              slab_u.at[b], sems_u[b]).wait()
        pltpu.make_async_copy(emb_i_hbm.at[pl.ds(0, _CH)],
                              slab_i.at[b], sems_i[b]).wait()

    def compute(k, b):
        su = lax.rem(plsc.load_gather(uidx_v, [k * _CH + lanes]), 8)
        si = lax.rem(plsc.load_gather(iidx_v, [k * _CH + lanes]), 8)
        bv = jnp.full((16,), b, jnp.int32)
        acc = jnp.zeros((16,), jnp.float32)
        for d in range(_D):
            dv = jnp.full((16,), d, jnp.int32)
            uv = plsc.load_gather(slab_u, [bv, lanes, su, dv])
            iv = plsc.load_gather(slab_i, [bv, lanes, si, dv])
            acc = acc + uv * iv
        dot_v[pl.ds(k * _CH, 16)] = acc

    fire(0, 0)

    def loop_body(k2, _):
        for half in range(_NBUF):
            k = k2 * _NBUF + half

            @pl.when(k + 1 < _NCHK)
            def _(k=k, half=half):
                fire(k + 1, (half + 1) % _NBUF)

            drain(half)
            compute(k, half)
        return ()

    lax.fori_loop(0, _NCHK // _NBUF, loop_body, ())
    pltpu.sync_copy(dot_v, out_hbm.at[pl.ds(base, _BPW)])


@functools.partial(
    pl.kernel,
    out_type=jax.ShapeDtypeStruct((_B,), jnp.float32),
    mesh=plsc.VectorSubcoreMesh(core_axis_name="c", subcore_axis_name="s",
                                num_cores=_NC, num_subcores=_NS),
    compiler_params=pltpu.CompilerParams(needs_layout_passes=False,
                                         use_tc_tiling_on_sc=False),
    scratch_types=[
        pltpu.VMEM((_BPW,), jnp.int32),
        pltpu.VMEM((_BPW,), jnp.int32),
        pltpu.VMEM((_BPW,), jnp.int32),
        pltpu.VMEM((_BPW,), jnp.int32),
        pltpu.VMEM((_NBUF, _CH, 8, _D), jnp.float32),
        pltpu.VMEM((_NBUF, _CH, 8, _D), jnp.float32),
        pltpu.VMEM((_BPW,), jnp.float32),
        [pltpu.SemaphoreType.DMA for _ in range(_NBUF)],
        [pltpu.SemaphoreType.DMA for _ in range(_NBUF)],
    ],
)
def _sc_dot(uidx_hbm, iidx_hbm, emb_u_hbm, emb_i_hbm, out_hbm,
            uidx_v, iidx_v, tidx_u, tidx_i, slab_u, slab_i, dot_v,
            sems_u, sems_i):
    _sc_body(uidx_hbm, iidx_hbm, emb_u_hbm, emb_i_hbm, out_hbm,
             uidx_v, iidx_v, tidx_u, tidx_i, slab_u, slab_i, dot_v,
             sems_u, sems_i)


def _tc_body(dot_ref, labels_ref, rating_ref, loss_ref):
    x = dot_ref[...]
    r = jax.nn.sigmoid(x)
    rating_ref[...] = r
    y = labels_ref[...]
    t = jnp.maximum(r, 0.0) - r * y + jnp.log1p(jnp.exp(-jnp.abs(r)))
    loss_ref[0, 0] = jnp.sum(t) / _B


def kernel(user_indices, item_indices, labels, emb_user, emb_item):
    emb_u3 = emb_user.reshape(125000, 8, _D)
    emb_i3 = emb_item.reshape(125000, 8, _D)
    dot = _sc_dot(user_indices, item_indices, emb_u3, emb_i3)

    rating2d, loss11 = pl.pallas_call(
        _tc_body,
        out_shape=[
            jax.ShapeDtypeStruct((_B // 128, 128), jnp.float32),
            jax.ShapeDtypeStruct((1, 1), jnp.float32),
        ],
        out_specs=[
            pl.BlockSpec(memory_space=pltpu.VMEM),
            pl.BlockSpec(memory_space=pltpu.SMEM),
        ],
    )(dot.reshape(_B // 128, 128), labels.reshape(_B // 128, 128))

    rating = rating2d.reshape(_B)
    loss = loss11.reshape(())
    return (loss, loss, rating, labels)


# speedup vs baseline: 2.3208x; 2.3208x over previous
import functools

import jax
import jax.numpy as jnp
from jax import lax
from jax.experimental import pallas as pl
from jax.experimental.pallas import tpu as pltpu
from jax.experimental.pallas import tpu_sc as plsc

_B = 16384
_D = 32
_NC = 2
_NS = 16
_NW = _NC * _NS
_BPW = _B // _NW
_CH = 16
_NCHK = _BPW // _CH
_NBUF = 2


def _sc_body(uidx_hbm, iidx_hbm, emb_u_hbm, emb_i_hbm, out_hbm,
             uidx_v, iidx_v, slab_u, slab_i, dot_v, sems_u, sems_i):
    c = lax.axis_index("c")
    s = lax.axis_index("s")
    wid = s * _NC + c
    base = wid * _BPW
    pltpu.sync_copy(uidx_hbm.at[pl.ds(base, _BPW)], uidx_v)
    pltpu.sync_copy(iidx_hbm.at[pl.ds(base, _BPW)], iidx_v)

    lanes = lax.iota(jnp.int32, 16)

    def fire(k, b):
        tu = lax.div(uidx_v[pl.ds(k * _CH, 16)], 8)
        ti = lax.div(iidx_v[pl.ds(k * _CH, 16)], 8)
        for i in range(_CH):
            pltpu.async_copy(emb_u_hbm.at[tu[i]],
                             slab_u.at[b, i], sems_u[b])
            pltpu.async_copy(emb_i_hbm.at[ti[i]],
                             slab_i.at[b, i], sems_i[b])

    def drain(b):
        pltpu.make_async_copy(emb_u_hbm.at[pl.ds(0, _CH)],
                              slab_u.at[b], sems_u[b]).wait()
        pltpu.make_async_copy(emb_i_hbm.at[pl.ds(0, _CH)],
                              slab_i.at[b], sems_i[b]).wait()

    def compute(k, b):
        su = lax.rem(plsc.load_gather(uidx_v, [k * _CH + lanes]), 8)
        si = lax.rem(plsc.load_gather(iidx_v, [k * _CH + lanes]), 8)
        bv = jnp.full((16,), b, jnp.int32)
        acc = jnp.zeros((16,), jnp.float32)
        for d in range(_D):
            dv = jnp.full((16,), d, jnp.int32)
            uv = plsc.load_gather(slab_u, [bv, lanes, su, dv])
            iv = plsc.load_gather(slab_i, [bv, lanes, si, dv])
            acc = acc + uv * iv
        dot_v[pl.ds(k * _CH, 16)] = acc

    fire(0, 0)

    def loop_body(k2, _):
        for half in range(_NBUF):
            k = k2 * _NBUF + half

            @pl.when(k + 1 < _NCHK)
            def _(k=k, half=half):
                fire(k + 1, (half + 1) % _NBUF)

            drain(half)
            compute(k, half)
        return ()

    lax.fori_loop(0, _NCHK // _NBUF, loop_body, ())
    pltpu.sync_copy(dot_v, out_hbm.at[pl.ds(base, _BPW)])


@functools.partial(
    pl.kernel,
    out_type=jax.ShapeDtypeStruct((_B,), jnp.float32),
    mesh=plsc.VectorSubcoreMesh(core_axis_name="c", subcore_axis_name="s",
                                num_cores=_NC, num_subcores=_NS),
    compiler_params=pltpu.CompilerParams(needs_layout_passes=False),
    scratch_types=[
        pltpu.VMEM((_BPW,), jnp.int32),
        pltpu.VMEM((_BPW,), jnp.int32),
        pltpu.VMEM((_NBUF, _CH, 8, _D), jnp.float32),
        pltpu.VMEM((_NBUF, _CH, 8, _D), jnp.float32),
        pltpu.VMEM((_BPW,), jnp.float32),
        [pltpu.SemaphoreType.DMA for _ in range(_NBUF)],
        [pltpu.SemaphoreType.DMA for _ in range(_NBUF)],
    ],
)
def _sc_dot(uidx_hbm, iidx_hbm, emb_u_hbm, emb_i_hbm, out_hbm,
            uidx_v, iidx_v, slab_u, slab_i, dot_v, sems_u, sems_i):
    _sc_body(uidx_hbm, iidx_hbm, emb_u_hbm, emb_i_hbm, out_hbm,
             uidx_v, iidx_v, slab_u, slab_i, dot_v, sems_u, sems_i)


def _tc_body(dot_ref, labels_ref, rating_ref, loss_ref):
    x = dot_ref[...]
    r = jax.nn.sigmoid(x)
    rating_ref[...] = r
    y = labels_ref[...]
    t = jnp.maximum(r, 0.0) - r * y + jnp.log1p(jnp.exp(-jnp.abs(r)))
    loss_ref[0, 0] = jnp.sum(t) / _B


def kernel(user_indices, item_indices, labels, emb_user, emb_item):
    emb_u3 = emb_user.reshape(125000, 8, _D)
    emb_i3 = emb_item.reshape(125000, 8, _D)
    dot = _sc_dot(user_indices, item_indices, emb_u3, emb_i3)

    rating2d, loss11 = pl.pallas_call(
        _tc_body,
        out_shape=[
            jax.ShapeDtypeStruct((_B // 128, 128), jnp.float32),
            jax.ShapeDtypeStruct((1, 1), jnp.float32),
        ],
        out_specs=[
            pl.BlockSpec(memory_space=pltpu.VMEM),
            pl.BlockSpec(memory_space=pltpu.SMEM),
        ],
    )(dot.reshape(_B // 128, 128), labels.reshape(_B // 128, 128))

    rating = rating2d.reshape(_B)
    loss = loss11.reshape(())
    return (loss, loss, rating, labels)


# speedup vs baseline: 2.5385x; 1.0938x over previous
import functools

import jax
import jax.numpy as jnp
from jax import lax
from jax.experimental import pallas as pl
from jax.experimental.pallas import tpu as pltpu
from jax.experimental.pallas import tpu_sc as plsc

_B = 16384
_D = 32
_NC = 2
_NS = 16
_NW = _NC * _NS
_BPW = _B // _NW
_CH = 16
_NCHK = _BPW // _CH
_NBUF = 2


def _sc_body(uidx_hbm, iidx_hbm, emb_u_hbm, emb_i_hbm, out_hbm,
             uidx_v, iidx_v, slab_u, slab_i, dot_v, sems_u, sems_i):
    c = lax.axis_index("c")
    s = lax.axis_index("s")
    wid = s * _NC + c
    base = wid * _BPW
    pltpu.sync_copy(uidx_hbm.at[pl.ds(base, _BPW)], uidx_v)
    pltpu.sync_copy(iidx_hbm.at[pl.ds(base, _BPW)], iidx_v)

    lanes = lax.iota(jnp.int32, 16)

    def fire(k, b):
        tu = uidx_v[pl.ds(k * _CH, 16)]
        ti = iidx_v[pl.ds(k * _CH, 16)]
        for i in range(_CH):
            pltpu.async_copy(emb_u_hbm.at[tu[i]],
                             slab_u.at[b, i], sems_u[b])
            pltpu.async_copy(emb_i_hbm.at[ti[i]],
                             slab_i.at[b, i], sems_i[b])

    def drain(b):
        pltpu.make_async_copy(emb_u_hbm.at[pl.ds(0, _CH)],
                              slab_u.at[b], sems_u[b]).wait()
        pltpu.make_async_copy(emb_i_hbm.at[pl.ds(0, _CH)],
                              slab_i.at[b], sems_i[b]).wait()

    def compute(k, b):
        bv = jnp.full((16,), b, jnp.int32)
        zv = jnp.zeros((16,), jnp.int32)
        acc = jnp.zeros((16,), jnp.float32)
        for d in range(_D):
            dv = jnp.full((16,), d, jnp.int32)
            uv = plsc.load_gather(slab_u, [bv, lanes, zv, dv])
            iv = plsc.load_gather(slab_i, [bv, lanes, zv, dv])
            acc = acc + uv * iv
        dot_v[pl.ds(k * _CH, 16)] = acc

    fire(0, 0)

    def loop_body(k2, _):
        for half in range(_NBUF):
            k = k2 * _NBUF + half

            @pl.when(k + 1 < _NCHK)
            def _(k=k, half=half):
                fire(k + 1, (half + 1) % _NBUF)

            drain(half)
            compute(k, half)
        return ()

    lax.fori_loop(0, _NCHK // _NBUF, loop_body, ())
    pltpu.sync_copy(dot_v, out_hbm.at[pl.ds(base, _BPW)])


@functools.partial(
    pl.kernel,
    out_type=jax.ShapeDtypeStruct((_B,), jnp.float32),
    mesh=plsc.VectorSubcoreMesh(core_axis_name="c", subcore_axis_name="s",
                                num_cores=_NC, num_subcores=_NS),
    compiler_params=pltpu.CompilerParams(needs_layout_passes=False),
    scratch_types=[
        pltpu.VMEM((_BPW,), jnp.int32),
        pltpu.VMEM((_BPW,), jnp.int32),
        pltpu.VMEM((_NBUF, _CH, 1, _D), jnp.float32),
        pltpu.VMEM((_NBUF, _CH, 1, _D), jnp.float32),
        pltpu.VMEM((_BPW,), jnp.float32),
        [pltpu.SemaphoreType.DMA for _ in range(_NBUF)],
        [pltpu.SemaphoreType.DMA for _ in range(_NBUF)],
    ],
)
def _sc_dot(uidx_hbm, iidx_hbm, emb_u_hbm, emb_i_hbm, out_hbm,
            uidx_v, iidx_v, slab_u, slab_i, dot_v, sems_u, sems_i):
    _sc_body(uidx_hbm, iidx_hbm, emb_u_hbm, emb_i_hbm, out_hbm,
             uidx_v, iidx_v, slab_u, slab_i, dot_v, sems_u, sems_i)


def _tc_body(dot_ref, labels_ref, rating_ref, loss_ref):
    x = dot_ref[...]
    r = jax.nn.sigmoid(x)
    rating_ref[...] = r
    y = labels_ref[...]
    t = jnp.maximum(r, 0.0) - r * y + jnp.log1p(jnp.exp(-jnp.abs(r)))
    loss_ref[0, 0] = jnp.sum(t) / _B


def kernel(user_indices, item_indices, labels, emb_user, emb_item):
    emb_u3 = emb_user.reshape(1000000, 1, _D)
    emb_i3 = emb_item.reshape(1000000, 1, _D)
    dot = _sc_dot(user_indices, item_indices, emb_u3, emb_i3)

    rating2d, loss11 = pl.pallas_call(
        _tc_body,
        out_shape=[
            jax.ShapeDtypeStruct((_B // 128, 128), jnp.float32),
            jax.ShapeDtypeStruct((1, 1), jnp.float32),
        ],
        out_specs=[
            pl.BlockSpec(memory_space=pltpu.VMEM),
            pl.BlockSpec(memory_space=pltpu.SMEM),
        ],
    )(dot.reshape(_B // 128, 128), labels.reshape(_B // 128, 128))

    rating = rating2d.reshape(_B)
    loss = loss11.reshape(())
    return (loss, loss, rating, labels)
